# baseline (speedup 1.0000x reference)
import functools

import jax
import jax.numpy as jnp
from jax import lax
from jax.experimental import pallas as pl
from jax.experimental.pallas import tpu as pltpu
from jax.experimental.pallas import tpu_sc as plsc

_NODE_DIM = 128
_CHUNK = 100
_N_WORKERS = 32
_TC_GRID = 10


def _gather_rows_sc(table, head_idx, tail_idx):
    n_chunks_half = head_idx.shape[0]
    mesh = plsc.VectorSubcoreMesh(core_axis_name="c", subcore_axis_name="s")
    row_ty = jax.ShapeDtypeStruct((n_chunks_half, _CHUNK, _NODE_DIM), jnp.float32)

    @functools.partial(
        pl.kernel,
        out_type=[row_ty, row_ty],
        mesh=mesh,
        scratch_types=[
            pltpu.VMEM((2, _CHUNK), jnp.int32),
            pltpu.VMEM((2, _CHUNK, _NODE_DIM), jnp.float32),
            pltpu.SemaphoreType.DMA,
        ],
    )
    def gather_kernel(table_hbm, head_hbm, tail_hbm, hout_hbm, tout_hbm,
                      idx_v, rows_v, sem):
        wid = lax.axis_index("s") * 2 + lax.axis_index("c")
        pltpu.sync_copy(head_hbm.at[pl.ds(wid, 1)], idx_v.at[pl.ds(0, 1)])
        pltpu.sync_copy(tail_hbm.at[pl.ds(wid, 1)], idx_v.at[pl.ds(1, 1)])
        copies = [
            pltpu.async_copy(table_hbm.at[idx_v.at[j]], rows_v.at[j], sem)
            for j in range(2)
        ]
        for cp in copies:
            cp.wait()
        pltpu.sync_copy(rows_v.at[pl.ds(0, 1)], hout_hbm.at[pl.ds(wid, 1)])
        pltpu.sync_copy(rows_v.at[pl.ds(1, 1)], tout_hbm.at[pl.ds(wid, 1)])

    return gather_kernel(table, head_idx, tail_idx)


def _predict_tc(head_rows, tail_rows, W_fc, b_fc2d, W_pred, b_pred2d):
    n_pairs = head_rows.shape[0]
    d = W_fc.shape[1]
    out_num = b_pred2d.shape[1]
    blk = n_pairs // _TC_GRID

    def body(rh_ref, rt_ref, wfc_ref, bfc_ref, wp_ref, bp_ref, out_ref):
        wfc = wfc_ref[...]
        bfc = bfc_ref[...]
        wp = wp_ref[...]
        g_h = jnp.maximum(
            jnp.dot(rh_ref[...], wfc, preferred_element_type=jnp.float32) + bfc,
            0.0)
        g_t = jnp.maximum(
            jnp.dot(rt_ref[...], wfc, preferred_element_type=jnp.float32) + bfc,
            0.0)
        out_ref[...] = (
            jnp.dot(g_h, wp[:d], preferred_element_type=jnp.float32)
            + jnp.dot(g_t, wp[d:], preferred_element_type=jnp.float32)
            + bp_ref[...]
        )

    return pl.pallas_call(
        body,
        grid=(_TC_GRID,),
        in_specs=[
            pl.BlockSpec((blk, d), lambda i: (i, 0)),
            pl.BlockSpec((blk, d), lambda i: (i, 0)),
            pl.BlockSpec((d, d), lambda i: (0, 0)),
            pl.BlockSpec((1, d), lambda i: (0, 0)),
            pl.BlockSpec((2 * d, out_num), lambda i: (0, 0)),
            pl.BlockSpec((1, out_num), lambda i: (0, 0)),
        ],
        out_specs=pl.BlockSpec((blk, out_num), lambda i: (i, 0)),
        out_shape=jax.ShapeDtypeStruct((n_pairs, out_num), jnp.float32),
    )(head_rows, tail_rows, W_fc, b_fc2d, W_pred, b_pred2d)


def kernel(node_features, edge_index, edge_features, head_ent_nodes,
           tail_ent_nodes, W_fc, b_fc, W_pred, b_pred):
    del edge_index, edge_features
    B, P = head_ent_nodes.shape
    out_num = b_pred.shape[0]
    node_dim = W_fc.shape[1]

    head_rows, tail_rows = _gather_rows_sc(node_features, head_ent_nodes,
                                           tail_ent_nodes)
    out = _predict_tc(head_rows.reshape(B * P, node_dim),
                      tail_rows.reshape(B * P, node_dim),
                      W_fc, b_fc.reshape(1, node_dim),
                      W_pred, b_pred.reshape(1, out_num))
    return out.reshape(B, P, out_num)


# speedup vs baseline: 1.1133x; 1.1133x over previous
import functools

import jax
import jax.numpy as jnp
from jax import lax
from jax.experimental import pallas as pl
from jax.experimental.pallas import tpu as pltpu
from jax.experimental.pallas import tpu_sc as plsc

_NODE_DIM = 128
_CHUNK = 100
_N_WORKERS = 32
_TC_GRID = 10


def _gather_rows_sc(table, head_idx, tail_idx):
    n_chunks_half = head_idx.shape[0]
    mesh = plsc.VectorSubcoreMesh(core_axis_name="c", subcore_axis_name="s")
    row_ty = jax.ShapeDtypeStruct((n_chunks_half, _CHUNK, _NODE_DIM), jnp.float32)

    @functools.partial(
        pl.kernel,
        out_type=[row_ty, row_ty],
        mesh=mesh,
        scratch_types=[
            pltpu.VMEM((2, _CHUNK), jnp.int32),
            pltpu.VMEM((2, _CHUNK, _NODE_DIM), jnp.float32),
            pltpu.SemaphoreType.DMA,
        ],
    )
    def gather_kernel(table_hbm, head_hbm, tail_hbm, hout_hbm, tout_hbm,
                      idx_v, rows_v, sem):
        wid = lax.axis_index("s") * 2 + lax.axis_index("c")
        pltpu.sync_copy(head_hbm.at[pl.ds(wid, 1)], idx_v.at[pl.ds(0, 1)])
        pltpu.sync_copy(tail_hbm.at[pl.ds(wid, 1)], idx_v.at[pl.ds(1, 1)])
        copies = [
            pltpu.async_copy(table_hbm.at[idx_v.at[j]], rows_v.at[j], sem)
            for j in range(2)
        ]
        for cp in copies:
            cp.wait()
        pltpu.sync_copy(rows_v.at[pl.ds(0, 1)], hout_hbm.at[pl.ds(wid, 1)])
        pltpu.sync_copy(rows_v.at[pl.ds(1, 1)], tout_hbm.at[pl.ds(wid, 1)])

    return gather_kernel(table, head_idx, tail_idx)


def _predict_tc(head_rows, tail_rows, W_fc, b_fc2d, W_pred, b_pred2d):
    n_pairs = head_rows.shape[0]
    d = W_fc.shape[1]
    out_num = b_pred2d.shape[1]
    blk = n_pairs // _TC_GRID

    def body(rh_ref, rt_ref, wfc_ref, bfc_ref, wp_ref, bp_ref, out_ref):
        wfc = wfc_ref[...]
        bfc = bfc_ref[...]
        wp = wp_ref[...]
        g_h = jnp.maximum(
            jnp.dot(rh_ref[...], wfc, preferred_element_type=jnp.float32) + bfc,
            0.0)
        g_t = jnp.maximum(
            jnp.dot(rt_ref[...], wfc, preferred_element_type=jnp.float32) + bfc,
            0.0)
        out_ref[...] = (
            jnp.dot(g_h, wp[:d], preferred_element_type=jnp.float32)
            + jnp.dot(g_t, wp[d:], preferred_element_type=jnp.float32)
            + bp_ref[...]
        )

    del blk
    return pl.pallas_call(
        body,
        out_shape=jax.ShapeDtypeStruct((n_pairs, out_num), jnp.float32),
    )(head_rows, tail_rows, W_fc, b_fc2d, W_pred, b_pred2d)


def kernel(node_features, edge_index, edge_features, head_ent_nodes,
           tail_ent_nodes, W_fc, b_fc, W_pred, b_pred):
    del edge_index, edge_features
    B, P = head_ent_nodes.shape
    out_num = b_pred.shape[0]
    node_dim = W_fc.shape[1]

    head_rows, tail_rows = _gather_rows_sc(node_features, head_ent_nodes,
                                           tail_ent_nodes)
    out = _predict_tc(head_rows.reshape(B * P, node_dim),
                      tail_rows.reshape(B * P, node_dim),
                      W_fc, b_fc.reshape(1, node_dim),
                      W_pred, b_pred.reshape(1, out_num))
    return out.reshape(B, P, out_num)


# speedup vs baseline: 1.3879x; 1.2467x over previous
import functools

import jax
import jax.numpy as jnp
from jax import lax
from jax.experimental import pallas as pl
from jax.experimental.pallas import tpu as pltpu
from jax.experimental.pallas import tpu_sc as plsc

_NODE_DIM = 128
_CHUNK = 100
_N_WORKERS = 32
_ROWS_PER_W = 200


def _gather_rows_sc(table, head_idx, tail_idx):
    n_rows = head_idx.size
    mesh = plsc.VectorSubcoreMesh(core_axis_name="c", subcore_axis_name="s")
    row_ty = jax.ShapeDtypeStruct((n_rows, _NODE_DIM), jnp.float32)

    @functools.partial(
        pl.kernel,
        out_type=[row_ty, row_ty],
        mesh=mesh,
        scratch_types=[
            pltpu.VMEM((2, _CHUNK), jnp.int32),
            pltpu.VMEM((_ROWS_PER_W, _NODE_DIM), jnp.float32),
            pltpu.SemaphoreType.DMA,
        ],
    )
    def gather_kernel(table_hbm, head_hbm, tail_hbm, hout_hbm, tout_hbm,
                      idx_v, rows_v, sem):
        wid = lax.axis_index("s") * 2 + lax.axis_index("c")

        def gather_half(idx_hbm, out_hbm, w):
            pltpu.sync_copy(idx_hbm.at[pl.ds(2 * w, 2)], idx_v)
            copies = [
                pltpu.async_copy(table_hbm.at[idx_v.at[j]],
                                 rows_v.at[pl.ds(j * _CHUNK, _CHUNK)], sem)
                for j in range(2)
            ]
            for cp in copies:
                cp.wait()
            pltpu.sync_copy(rows_v, out_hbm.at[pl.ds(w * _ROWS_PER_W,
                                                     _ROWS_PER_W)])

        @pl.when(wid < _N_WORKERS // 2)
        def _():
            gather_half(head_hbm, hout_hbm, wid)

        @pl.when(wid >= _N_WORKERS // 2)
        def _():
            gather_half(tail_hbm, tout_hbm, wid - _N_WORKERS // 2)

    return gather_kernel(table, head_idx, tail_idx)


def _predict_tc(head_rows, tail_rows, W_fc, b_fc2d, W_pred, b_pred2d, B, P):
    d = W_fc.shape[1]
    out_num = b_pred2d.shape[1]

    def body(rh_ref, rt_ref, wfc_ref, bfc_ref, wp_ref, bp_ref, out_ref):
        wfc = wfc_ref[...]
        bfc = bfc_ref[...]
        wp = wp_ref[...]
        g_h = jnp.maximum(
            jnp.dot(rh_ref[...], wfc, preferred_element_type=jnp.float32) + bfc,
            0.0)
        g_t = jnp.maximum(
            jnp.dot(rt_ref[...], wfc, preferred_element_type=jnp.float32) + bfc,
            0.0)
        res = (
            jnp.dot(g_h, wp[:d], preferred_element_type=jnp.float32)
            + jnp.dot(g_t, wp[d:], preferred_element_type=jnp.float32)
            + bp_ref[...]
        )
        out_ref[...] = res.reshape(B, P, out_num)

    return pl.pallas_call(
        body,
        out_shape=jax.ShapeDtypeStruct((B, P, out_num), jnp.float32),
    )(head_rows, tail_rows, W_fc, b_fc2d, W_pred, b_pred2d)


def kernel(node_features, edge_index, edge_features, head_ent_nodes,
           tail_ent_nodes, W_fc, b_fc, W_pred, b_pred):
    del edge_index, edge_features
    B, P = head_ent_nodes.shape
    out_num = b_pred.shape[0]
    node_dim = W_fc.shape[1]

    head_rows, tail_rows = _gather_rows_sc(node_features, head_ent_nodes,
                                           tail_ent_nodes)
    return _predict_tc(head_rows, tail_rows, W_fc, b_fc.reshape(1, node_dim),
                       W_pred, b_pred.reshape(1, out_num), B, P)
